# baseline (device time: 12992 ns/iter reference)
import jax
import jax.numpy as jnp
from jax import lax
from jax.experimental import pallas as pl
from jax.experimental.pallas import tpu as pltpu

N_COLS_GLOBAL = 2048
BM = 512
SYNC = 3


def kernel(x):
    m, n = x.shape
    nsteps = m // BM

    def body(x_ref, out_ref, acc_ref, comm_ref, send_sems, recv_sems):
        i = pl.program_id(0)
        my_x = lax.axis_index("x")
        my_y = lax.axis_index("y")
        peer = (my_x, 1 - my_y)
        barrier_sem = pltpu.get_barrier_semaphore()

        @pl.when(i == 0)
        def _():
            pl.semaphore_signal(
                barrier_sem, inc=1,
                device_id=peer, device_id_type=pl.DeviceIdType.MESH,
            )

        t = x_ref[:, 0:128]
        for c in range(1, n // 128):
            t = t + x_ref[:, c * 128 : (c + 1) * 128]
        acc_ref[pl.ds(i, 1), :] = jnp.sum(t.T, axis=0, keepdims=True)

        @pl.when(i == SYNC)
        def _():
            pl.semaphore_wait(barrier_sem, 1)
            bulk = pltpu.make_async_remote_copy(
                src_ref=acc_ref.at[pl.ds(0, SYNC + 1)],
                dst_ref=comm_ref.at[pl.ds(0, SYNC + 1)],
                send_sem=send_sems.at[0],
                recv_sem=recv_sems.at[0],
                device_id=peer,
                device_id_type=pl.DeviceIdType.MESH,
            )
            bulk.start()

        @pl.when(i > SYNC)
        def _():
            rdma = pltpu.make_async_remote_copy(
                src_ref=acc_ref.at[pl.ds(i, 1)],
                dst_ref=comm_ref.at[pl.ds(i, 1)],
                send_sem=send_sems.at[i],
                recv_sem=recv_sems.at[i],
                device_id=peer,
                device_id_type=pl.DeviceIdType.MESH,
            )
            rdma.start()

        @pl.when(i == nsteps - 1)
        def _():
            at = acc_ref[:, :].T

            drains = [(0, SYNC + 1)] + [(j, 1) for j in range(SYNC + 1, nsteps)]
            for slot, size in drains:
                d = pltpu.make_async_remote_copy(
                    src_ref=acc_ref.at[pl.ds(slot, size)],
                    dst_ref=comm_ref.at[pl.ds(slot, size)],
                    send_sem=send_sems.at[slot],
                    recv_sem=recv_sems.at[slot],
                    device_id=peer,
                    device_id_type=pl.DeviceIdType.MESH,
                )
                d.wait_send()
                d.wait_recv()

            ct = (at + comm_ref[:, :].T) * (1.0 / N_COLS_GLOBAL)
            for j in range(nsteps):
                out_ref[pl.ds(j * BM, BM), :] = ct[:, j : j + 1]

    return pl.pallas_call(
        body,
        grid=(nsteps,),
        out_shape=jax.ShapeDtypeStruct((m, 1), jnp.float32),
        in_specs=[
            pl.BlockSpec((BM, n), lambda i: (i, 0), memory_space=pltpu.VMEM)
        ],
        out_specs=pl.BlockSpec((m, 1), lambda i: (0, 0), memory_space=pltpu.VMEM),
        scratch_shapes=[
            pltpu.VMEM((nsteps, BM), jnp.float32),
            pltpu.VMEM((nsteps, BM), jnp.float32),
            pltpu.SemaphoreType.DMA((nsteps,)),
            pltpu.SemaphoreType.DMA((nsteps,)),
        ],
        compiler_params=pltpu.CompilerParams(
            collective_id=0,
            dimension_semantics=("arbitrary",),
        ),
    )(x)


# device time: 12708 ns/iter; 1.0223x vs baseline; 1.0223x over previous
import jax
import jax.numpy as jnp
from jax import lax
from jax.experimental import pallas as pl
from jax.experimental.pallas import tpu as pltpu

N_COLS_GLOBAL = 2048
BM = 512


def kernel(x):
    m, n = x.shape
    nsteps = m // BM
    half = nsteps // 2

    def body(x_ref, out_ref, acc_ref, comm_ref, send_sems, recv_sems):
        i = pl.program_id(0)
        my_x = lax.axis_index("x")
        my_y = lax.axis_index("y")
        peer = (my_x, 1 - my_y)

        @pl.when(i == 0)
        def _():
            barrier_sem = pltpu.get_barrier_semaphore()
            pl.semaphore_signal(
                barrier_sem, inc=1,
                device_id=peer, device_id_type=pl.DeviceIdType.MESH,
            )
            pl.semaphore_wait(barrier_sem, 1)

        @pl.when(i < nsteps)
        def _():
            t = x_ref[:, 0:128]
            for c in range(1, n // 128):
                t = t + x_ref[:, c * 128 : (c + 1) * 128]
            acc_ref[pl.ds(i, 1), :] = jnp.sum(t.T, axis=0, keepdims=True)

            rdma = pltpu.make_async_remote_copy(
                src_ref=acc_ref.at[pl.ds(i, 1)],
                dst_ref=comm_ref.at[pl.ds(i, 1)],
                send_sem=send_sems.at[i],
                recv_sem=recv_sems.at[i],
                device_id=peer,
                device_id_type=pl.DeviceIdType.MESH,
            )
            rdma.start()

        def drain(slots):
            for j in slots:
                d = pltpu.make_async_remote_copy(
                    src_ref=acc_ref.at[pl.ds(j, 1)],
                    dst_ref=comm_ref.at[pl.ds(j, 1)],
                    send_sem=send_sems.at[j],
                    recv_sem=recv_sems.at[j],
                    device_id=peer,
                    device_id_type=pl.DeviceIdType.MESH,
                )
                d.wait_send()
                d.wait_recv()

        def combine_half(lo):
            ct = (
                acc_ref[pl.ds(lo, half), :] + comm_ref[pl.ds(lo, half), :]
            ).T * (1.0 / N_COLS_GLOBAL)
            for j in range(half):
                out_ref[pl.ds(j * BM, BM), :] = ct[:, j : j + 1]

        @pl.when(i == nsteps - 1)
        def _():
            drain(range(half))
            combine_half(0)

        @pl.when(i == nsteps)
        def _():
            drain(range(half, nsteps))
            combine_half(half)

    return pl.pallas_call(
        body,
        grid=(nsteps + 1,),
        out_shape=jax.ShapeDtypeStruct((m, 1), jnp.float32),
        in_specs=[
            pl.BlockSpec(
                (BM, n),
                lambda i: (jnp.minimum(i, nsteps - 1), 0),
                memory_space=pltpu.VMEM,
            )
        ],
        out_specs=pl.BlockSpec(
            (half * BM, 1),
            lambda i: (jnp.where(i < nsteps, 0, 1), 0),
            memory_space=pltpu.VMEM,
        ),
        scratch_shapes=[
            pltpu.VMEM((nsteps, BM), jnp.float32),
            pltpu.VMEM((nsteps, BM), jnp.float32),
            pltpu.SemaphoreType.DMA((nsteps,)),
            pltpu.SemaphoreType.DMA((nsteps,)),
        ],
        compiler_params=pltpu.CompilerParams(
            collective_id=0,
            dimension_semantics=("arbitrary",),
        ),
    )(x)
